# baseline (speedup 1.0000x reference)
import functools

import jax
import jax.numpy as jnp
from jax import lax
from jax.experimental import pallas as pl
from jax.experimental.pallas import tpu as pltpu
from jax.experimental.pallas import tpu_sc as plsc

N_NODES = 10000
N_EDGES = 320000
D_FEAT = 128

NC = 2
NS = 16
NW = NC * NS

CHUNK = 88
NCHUNKS = 114
EDGES_PER_TILE = NCHUNKS * CHUNK
N_EPAD = EDGES_PER_TILE * NW
N_H = N_NODES + 8
NOCTETS = (NCHUNKS + 7) // 8
RCHUNK = 80
NRCHUNKS = N_NODES // RCHUNK
RROUNDS = (NRCHUNKS + NS - 1) // NS


def _mm_disc_body(x_ref, w_ref, b_ref, o_ref):
    o_ref[pl.ds(0, N_NODES), :] = (
        jnp.dot(x_ref[...], w_ref[...], preferred_element_type=jnp.float32)
        + b_ref[...]
    )
    o_ref[pl.ds(N_NODES, 8), :] = jnp.zeros((8, D_FEAT), jnp.float32)


def _mm_inc_body(p_ref, w_ref, b_ref, o_ref):
    s = p_ref[0] + p_ref[1]
    o_ref[...] = jnp.maximum(
        jnp.dot(s, w_ref[...], preferred_element_type=jnp.float32) + b_ref[...],
        0.0,
    )


_sc_mesh = plsc.VectorSubcoreMesh(core_axis_name="c", subcore_axis_name="s")

_idx_scratch = [pltpu.VMEM((CHUNK,), jnp.int32) for _ in range(16)]


@functools.partial(
    pl.kernel,
    out_type=jax.ShapeDtypeStruct((NC, N_NODES, D_FEAT), jnp.float32),
    mesh=_sc_mesh,
    scratch_types=[pltpu.VMEM((4, CHUNK, D_FEAT), jnp.float32)]
    + _idx_scratch
    + [
        pltpu.VMEM_SHARED((N_NODES, D_FEAT), jnp.float32),
        pltpu.SemaphoreType.DMA,
        pltpu.SemaphoreType.DMA,
        pltpu.SemaphoreType.DMA,
        pltpu.SemaphoreType.DMA,
    ],
)
def _sc_scatter(h_hbm, src_hbm, dst_hbm, out_hbm, rows4,
                s0, s1, s2, s3, s4, s5, s6, s7,
                d0, d1, d2, d3, d4, d5, d6, d7,
                acc_sh, ism, idm, gsem, ssem):
    c = lax.axis_index("c")
    s = lax.axis_index("s")
    wid = c * NS + s
    ebase = wid * EDGES_PER_TILE
    srcb = [s0, s1, s2, s3, s4, s5, s6, s7]
    dstb = [d0, d1, d2, d3, d4, d5, d6, d7]

    def _zrow(i, carry):
        for j in range(D_FEAT // 16):
            rows4[0, i, pl.ds(j * 16, 16)] = jnp.zeros((16,), jnp.float32)
        return carry

    lax.fori_loop(0, CHUNK, _zrow, 0)

    def _zcopy(k, carry):
        chunk = k * NS + s

        @pl.when(chunk < NRCHUNKS)
        def _():
            pltpu.sync_copy(
                rows4.at[0].at[pl.ds(0, RCHUNK)],
                acc_sh.at[pl.ds(chunk * RCHUNK, RCHUNK)],
            )

        return carry

    lax.fori_loop(0, RROUNDS, _zcopy, 0)

    for t in range(7):
        pltpu.async_copy(src_hbm.at[pl.ds(ebase + t * CHUNK, CHUNK)], srcb[t], ism)
        pltpu.async_copy(dst_hbm.at[pl.ds(ebase + t * CHUNK, CHUNK)], dstb[t], idm)
    for t in range(3):
        pltpu.make_async_copy(src_hbm.at[pl.ds(0, CHUNK)], srcb[t], ism).wait()
        pltpu.async_copy(h_hbm.at[srcb[t]], rows4.at[t], gsem)
    plsc.subcore_barrier()

    def _octet(k, carry):
        for j in range(8):
            m = k * 8 + j

            @pl.when(m < NCHUNKS)
            def _(j=j, m=m):
                pltpu.make_async_copy(h_hbm.at[srcb[j]], rows4.at[j % 4], gsem).wait()
                pltpu.make_async_copy(dst_hbm.at[pl.ds(0, CHUNK)], dstb[j], idm).wait()
                pltpu.async_copy(rows4.at[j % 4], acc_sh.at[dstb[j]], ssem, add=True)

                @pl.when(m > 0)
                def _():
                    pltpu.make_async_copy(rows4.at[0], acc_sh.at[dstb[0]], ssem).wait()

                @pl.when(m + 7 < NCHUNKS)
                def _(j=j, m=m):
                    off = ebase + (m + 7) * CHUNK
                    pltpu.async_copy(src_hbm.at[pl.ds(off, CHUNK)], srcb[(j + 7) % 8], ism)
                    pltpu.async_copy(dst_hbm.at[pl.ds(off, CHUNK)], dstb[(j + 7) % 8], idm)

                @pl.when(m + 3 < NCHUNKS)
                def _(j=j, m=m):
                    pltpu.make_async_copy(src_hbm.at[pl.ds(0, CHUNK)], srcb[(j + 3) % 8], ism).wait()
                    pltpu.async_copy(h_hbm.at[srcb[(j + 3) % 8]], rows4.at[(j + 3) % 4], gsem)

        return carry

    lax.fori_loop(0, NOCTETS, _octet, 0)
    pltpu.make_async_copy(rows4.at[0], acc_sh.at[dstb[0]], ssem).wait()
    plsc.subcore_barrier()

    def _wcopy(k, carry):
        chunk = k * NS + s

        @pl.when(chunk < NRCHUNKS)
        def _():
            pltpu.sync_copy(
                acc_sh.at[pl.ds(chunk * RCHUNK, RCHUNK)],
                out_hbm.at[c].at[pl.ds(chunk * RCHUNK, RCHUNK)],
            )

        return carry

    lax.fori_loop(0, RROUNDS, _wcopy, 0)


_BM = 2000
_GRID = N_NODES // _BM


@jax.jit
def kernel(x, edge_index, W_disc, b_disc, W_inc, b_inc):
    npad = N_EPAD - N_EDGES
    src = jnp.concatenate(
        [edge_index[0].astype(jnp.int32),
         jnp.full((npad,), N_NODES, jnp.int32)]
    )
    dst = jnp.concatenate(
        [edge_index[1].astype(jnp.int32), jnp.zeros((npad,), jnp.int32)]
    )

    h = pl.pallas_call(
        _mm_disc_body,
        out_shape=jax.ShapeDtypeStruct((N_H, D_FEAT), jnp.float32),
    )(x, W_disc, b_disc.reshape(1, D_FEAT))

    partials = _sc_scatter(h, src, dst)

    out = pl.pallas_call(
        _mm_inc_body,
        out_shape=jax.ShapeDtypeStruct((N_NODES, D_FEAT), jnp.float32),
    )(partials, W_inc, b_inc.reshape(1, D_FEAT))
    return out


# speedup vs baseline: 1.2978x; 1.2978x over previous
import functools

import jax
import jax.numpy as jnp
from jax import lax
from jax.experimental import pallas as pl
from jax.experimental.pallas import tpu as pltpu
from jax.experimental.pallas import tpu_sc as plsc

N_NODES = 10000
N_EDGES = 320000
D_FEAT = 128

NC = 2
NS = 16
NW = NC * NS

CHUNK = 88
NCHUNKS = 114
EDGES_PER_TILE = NCHUNKS * CHUNK
N_EPAD = EDGES_PER_TILE * NW
N_H = N_NODES + 8
NOCTETS = (NCHUNKS + 7) // 8
RCHUNK = 80
NRCHUNKS = N_NODES // RCHUNK
RROUNDS = (NRCHUNKS + NS - 1) // NS


def _mm_disc_body(x_ref, w_ref, b_ref, o_ref):
    o_ref[pl.ds(0, N_NODES), :] = (
        jnp.dot(x_ref[...], w_ref[...], preferred_element_type=jnp.float32)
        + b_ref[...]
    )
    o_ref[pl.ds(N_NODES, 8), :] = jnp.zeros((8, D_FEAT), jnp.float32)


def _mm_inc_body(p_ref, w_ref, b_ref, o_ref):
    s = p_ref[0] + p_ref[1]
    o_ref[...] = jnp.maximum(
        jnp.dot(s, w_ref[...], preferred_element_type=jnp.float32) + b_ref[...],
        0.0,
    )


_sc_mesh = plsc.VectorSubcoreMesh(core_axis_name="c", subcore_axis_name="s")

_idx_scratch = [pltpu.VMEM((CHUNK,), jnp.int32) for _ in range(16)]


@functools.partial(
    pl.kernel,
    out_type=jax.ShapeDtypeStruct((NC, N_NODES, D_FEAT), jnp.float32),
    mesh=_sc_mesh,
    scratch_types=[pltpu.VMEM((4, CHUNK, D_FEAT), jnp.float32)]
    + _idx_scratch
    + [
        pltpu.VMEM_SHARED((N_NODES, D_FEAT), jnp.float32),
        pltpu.SemaphoreType.DMA,
        pltpu.SemaphoreType.DMA,
        pltpu.SemaphoreType.DMA,
        pltpu.SemaphoreType.DMA,
    ],
)
def _sc_scatter(h_hbm, src_hbm, dst_hbm, out_hbm, rows4,
                s0, s1, s2, s3, s4, s5, s6, s7,
                d0, d1, d2, d3, d4, d5, d6, d7,
                acc_sh, ism, idm, gsem, ssem):
    c = lax.axis_index("c")
    s = lax.axis_index("s")
    wid = c * NS + s
    ebase = wid * EDGES_PER_TILE
    srcb = [s0, s1, s2, s3, s4, s5, s6, s7]
    dstb = [d0, d1, d2, d3, d4, d5, d6, d7]

    def _zrow(i, carry):
        for j in range(D_FEAT // 16):
            rows4[0, i, pl.ds(j * 16, 16)] = jnp.zeros((16,), jnp.float32)
        return carry

    lax.fori_loop(0, CHUNK, _zrow, 0)

    def _zcopy(k, carry):
        chunk = k * NS + s

        @pl.when(chunk < NRCHUNKS)
        def _():
            pltpu.sync_copy(
                rows4.at[0].at[pl.ds(0, RCHUNK)],
                acc_sh.at[pl.ds(chunk * RCHUNK, RCHUNK)],
            )

        return carry

    lax.fori_loop(0, RROUNDS, _zcopy, 0)

    for t in range(7):
        pltpu.async_copy(src_hbm.at[pl.ds(ebase + t * CHUNK, CHUNK)], srcb[t], ism)
        pltpu.async_copy(dst_hbm.at[pl.ds(ebase + t * CHUNK, CHUNK)], dstb[t], idm)
    for t in range(3):
        pltpu.make_async_copy(src_hbm.at[pl.ds(0, CHUNK)], srcb[t], ism).wait()
        pltpu.async_copy(h_hbm.at[srcb[t]], rows4.at[t], gsem)
    plsc.subcore_barrier()

    def _octet(k, carry):
        for j in range(8):
            m = k * 8 + j

            @pl.when(m < NCHUNKS)
            def _(j=j, m=m):
                pltpu.make_async_copy(h_hbm.at[srcb[j]], rows4.at[j % 4], gsem).wait()
                pltpu.make_async_copy(dst_hbm.at[pl.ds(0, CHUNK)], dstb[j], idm).wait()
                pltpu.async_copy(rows4.at[j % 4], acc_sh.at[dstb[j]], ssem, add=True)

                @pl.when(m > 0)
                def _():
                    pltpu.make_async_copy(rows4.at[0], acc_sh.at[dstb[0]], ssem).wait()

                @pl.when(m + 7 < NCHUNKS)
                def _(j=j, m=m):
                    off = ebase + (m + 7) * CHUNK
                    pltpu.async_copy(src_hbm.at[pl.ds(off, CHUNK)], srcb[(j + 7) % 8], ism)
                    pltpu.async_copy(dst_hbm.at[pl.ds(off, CHUNK)], dstb[(j + 7) % 8], idm)

                @pl.when(m + 3 < NCHUNKS)
                def _(j=j, m=m):
                    pltpu.make_async_copy(src_hbm.at[pl.ds(0, CHUNK)], srcb[(j + 3) % 8], ism).wait()
                    pltpu.async_copy(h_hbm.at[srcb[(j + 3) % 8]], rows4.at[(j + 3) % 4], gsem)

        return carry

    lax.fori_loop(0, NOCTETS, _octet, 0)
    pltpu.make_async_copy(rows4.at[0], acc_sh.at[dstb[0]], ssem).wait()
    plsc.subcore_barrier()

    def _wcopy(k, carry):
        chunk = k * NS + s

        @pl.when(chunk < NRCHUNKS)
        def _():
            pltpu.sync_copy(
                acc_sh.at[pl.ds(chunk * RCHUNK, RCHUNK)],
                out_hbm.at[c].at[pl.ds(chunk * RCHUNK, RCHUNK)],
            )

        return carry

    lax.fori_loop(0, RROUNDS, _wcopy, 0)


_BM = 2000
_GRID = N_NODES // _BM


@jax.jit
def kernel(x, edge_index, W_disc, b_disc, W_inc, b_inc):
    npad = N_EPAD - N_EDGES
    pad_ids = jnp.arange(npad, dtype=jnp.int32)
    src = jnp.concatenate(
        [edge_index[0].astype(jnp.int32), N_NODES + (pad_ids % 8)]
    )
    dst = jnp.concatenate(
        [edge_index[1].astype(jnp.int32), (pad_ids * 97) % N_NODES]
    )

    h = pl.pallas_call(
        _mm_disc_body,
        out_shape=jax.ShapeDtypeStruct((N_H, D_FEAT), jnp.float32),
    )(x, W_disc, b_disc.reshape(1, D_FEAT))

    partials = _sc_scatter(h, src, dst)

    out = pl.pallas_call(
        _mm_inc_body,
        out_shape=jax.ShapeDtypeStruct((N_NODES, D_FEAT), jnp.float32),
    )(partials, W_inc, b_inc.reshape(1, D_FEAT))
    return out


# speedup vs baseline: 1.3325x; 1.0268x over previous
import functools

import jax
import jax.numpy as jnp
from jax import lax
from jax.experimental import pallas as pl
from jax.experimental.pallas import tpu as pltpu
from jax.experimental.pallas import tpu_sc as plsc

N_NODES = 10000
N_EDGES = 320000
D_FEAT = 128

NC = 2
NS = 16
NW = NC * NS

EDGES_PER_TILE = N_EDGES // NW
CHUNK = 80
NCHUNKS = EDGES_PER_TILE // CHUNK
NOCTETS = (NCHUNKS + 7) // 8
RCHUNK = 80
NRCHUNKS = N_NODES // RCHUNK
RROUNDS = (NRCHUNKS + NS - 1) // NS


def _mm_disc_body(x_ref, w_ref, b_ref, o_ref):
    o_ref[...] = (
        jnp.dot(x_ref[...], w_ref[...], preferred_element_type=jnp.float32)
        + b_ref[...]
    )


def _mm_inc_body(p_ref, w_ref, b_ref, o_ref):
    s = p_ref[0] + p_ref[1]
    o_ref[...] = jnp.maximum(
        jnp.dot(s, w_ref[...], preferred_element_type=jnp.float32) + b_ref[...],
        0.0,
    )


_sc_mesh = plsc.VectorSubcoreMesh(core_axis_name="c", subcore_axis_name="s")

_idx_scratch = [pltpu.VMEM((CHUNK,), jnp.int32) for _ in range(16)]


@functools.partial(
    pl.kernel,
    out_type=jax.ShapeDtypeStruct((NC, N_NODES, D_FEAT), jnp.float32),
    mesh=_sc_mesh,
    scratch_types=[pltpu.VMEM((4, CHUNK, D_FEAT), jnp.float32)]
    + _idx_scratch
    + [
        pltpu.VMEM_SHARED((N_NODES, D_FEAT), jnp.float32),
        pltpu.SemaphoreType.DMA,
        pltpu.SemaphoreType.DMA,
        pltpu.SemaphoreType.DMA,
        pltpu.SemaphoreType.DMA,
    ],
)
def _sc_scatter(h_hbm, src_hbm, dst_hbm, out_hbm, rows4,
                s0, s1, s2, s3, s4, s5, s6, s7,
                d0, d1, d2, d3, d4, d5, d6, d7,
                acc_sh, ism, idm, gsem, ssem):
    c = lax.axis_index("c")
    s = lax.axis_index("s")
    wid = c * NS + s
    ebase = wid * EDGES_PER_TILE
    srcb = [s0, s1, s2, s3, s4, s5, s6, s7]
    dstb = [d0, d1, d2, d3, d4, d5, d6, d7]

    def _zrow(i, carry):
        for j in range(D_FEAT // 16):
            rows4[0, i, pl.ds(j * 16, 16)] = jnp.zeros((16,), jnp.float32)
        return carry

    lax.fori_loop(0, CHUNK, _zrow, 0)

    def _zcopy(k, carry):
        chunk = k * NS + s

        @pl.when(chunk < NRCHUNKS)
        def _():
            pltpu.sync_copy(
                rows4.at[0], acc_sh.at[pl.ds(chunk * RCHUNK, RCHUNK)]
            )

        return carry

    lax.fori_loop(0, RROUNDS, _zcopy, 0)

    for t in range(7):
        pltpu.async_copy(src_hbm.at[pl.ds(ebase + t * CHUNK, CHUNK)], srcb[t], ism)
        pltpu.async_copy(dst_hbm.at[pl.ds(ebase + t * CHUNK, CHUNK)], dstb[t], idm)
    for t in range(3):
        pltpu.make_async_copy(src_hbm.at[pl.ds(0, CHUNK)], srcb[t], ism).wait()
        pltpu.async_copy(h_hbm.at[srcb[t]], rows4.at[t], gsem)
    plsc.subcore_barrier()

    def _octet(k, carry):
        for j in range(8):
            m = k * 8 + j

            @pl.when(m < NCHUNKS)
            def _(j=j, m=m):
                pltpu.make_async_copy(h_hbm.at[srcb[j]], rows4.at[j % 4], gsem).wait()
                pltpu.make_async_copy(dst_hbm.at[pl.ds(0, CHUNK)], dstb[j], idm).wait()
                pltpu.async_copy(rows4.at[j % 4], acc_sh.at[dstb[j]], ssem, add=True)

                @pl.when(m > 0)
                def _():
                    pltpu.make_async_copy(rows4.at[0], acc_sh.at[dstb[0]], ssem).wait()

                @pl.when(m + 7 < NCHUNKS)
                def _(j=j, m=m):
                    off = ebase + (m + 7) * CHUNK
                    pltpu.async_copy(src_hbm.at[pl.ds(off, CHUNK)], srcb[(j + 7) % 8], ism)
                    pltpu.async_copy(dst_hbm.at[pl.ds(off, CHUNK)], dstb[(j + 7) % 8], idm)

                @pl.when(m + 3 < NCHUNKS)
                def _(j=j, m=m):
                    pltpu.make_async_copy(src_hbm.at[pl.ds(0, CHUNK)], srcb[(j + 3) % 8], ism).wait()
                    pltpu.async_copy(h_hbm.at[srcb[(j + 3) % 8]], rows4.at[(j + 3) % 4], gsem)

        return carry

    lax.fori_loop(0, NOCTETS, _octet, 0)
    pltpu.make_async_copy(rows4.at[0], acc_sh.at[dstb[0]], ssem).wait()
    plsc.subcore_barrier()

    def _wcopy(k, carry):
        chunk = k * NS + s

        @pl.when(chunk < NRCHUNKS)
        def _():
            pltpu.sync_copy(
                acc_sh.at[pl.ds(chunk * RCHUNK, RCHUNK)],
                out_hbm.at[c].at[pl.ds(chunk * RCHUNK, RCHUNK)],
            )

        return carry

    lax.fori_loop(0, RROUNDS, _wcopy, 0)


@jax.jit
def kernel(x, edge_index, W_disc, b_disc, W_inc, b_inc):
    src = edge_index[0].astype(jnp.int32)
    dst = edge_index[1].astype(jnp.int32)

    h = pl.pallas_call(
        _mm_disc_body,
        out_shape=jax.ShapeDtypeStruct((N_NODES, D_FEAT), jnp.float32),
    )(x, W_disc, b_disc.reshape(1, D_FEAT))

    partials = _sc_scatter(h, src, dst)

    out = pl.pallas_call(
        _mm_inc_body,
        out_shape=jax.ShapeDtypeStruct((N_NODES, D_FEAT), jnp.float32),
    )(partials, W_inc, b_inc.reshape(1, D_FEAT))
    return out
